# baseline (speedup 1.0000x reference)
import functools

import jax
import jax.numpy as jnp
from jax import lax
from jax.experimental import pallas as pl
from jax.experimental.pallas import tpu as pltpu
from jax.experimental.pallas import tpu_sc as plsc

B, S, H = 64, 512, 768
HW = H // 2
V = 30522
EPS = 1e-12

NC, NS = 2, 16
NW = NC * NS
TOK = B * S
CHUNK = 32
NPIPE = 4
SEQ_PER_PIPE = B // NPIPE
TOK_PER_PIPE = TOK // NPIPE
TPW = TOK_PER_PIPE // NW
NCHUNK = TPW // CHUNK
LANES = 16
PACKS_PER_ROW = HW // LANES


@functools.lru_cache(maxsize=1)
def _sc_gather_fn():
  mesh = plsc.VectorSubcoreMesh(core_axis_name="c", subcore_axis_name="s",
                                num_cores=NC, num_subcores=NS)

  @functools.partial(
      pl.kernel,
      mesh=mesh,
      out_type=jax.ShapeDtypeStruct((TOK_PER_PIPE, HW), jnp.int32),
      scratch_types=[
          pltpu.VMEM((NCHUNK, CHUNK), jnp.int32),
          pltpu.VMEM((CHUNK, H), jnp.int32),
          pltpu.VMEM((CHUNK, H), jnp.int32),
          pltpu.VMEM((CHUNK, HW), jnp.int32),
          pltpu.VMEM((CHUNK, HW), jnp.int32),
          pltpu.SemaphoreType.DMA,
          pltpu.SemaphoreType.DMA,
          pltpu.SemaphoreType.DMA,
          pltpu.SemaphoreType.DMA,
      ],
  )
  def sc_gather(word_hbm, ids_hbm, out_hbm, idx_v, f0, f1, p0, p1,
                g0, g1, o0, o1):
    wid = lax.axis_index("s") * NC + lax.axis_index("c")
    base = wid * TPW
    fbufs = (f0, f1)
    pbufs = (p0, p1)
    gsems = (g0, g1)
    osems = (o0, o1)
    pltpu.sync_copy(ids_hbm.at[wid], idx_v)

    def pack_chunk(fbuf, pbuf):
      def row_body(r, _):
        for j in range(PACKS_PER_ROW):
          a = fbuf[r, pl.ds(LANES * j, LANES)]
          bv = fbuf[r, pl.ds(HW + LANES * j, LANES)]
          ar = a + jnp.int32(0x8000)
          br = bv + jnp.int32(0x8000)
          word = lax.shift_right_logical(ar, 16) | (br & jnp.int32(-65536))
          pbuf[r, pl.ds(LANES * j, LANES)] = word
        return 0
      lax.fori_loop(0, CHUNK, row_body, 0)

    gh = [None] * NCHUNK
    sh = [None] * NCHUNK
    gh[0] = pltpu.async_copy(word_hbm.at[idx_v.at[0]], fbufs[0], gsems[0])
    for c in range(NCHUNK):
      b = c & 1
      gh[c].wait()
      if c + 1 < NCHUNK:
        gh[c + 1] = pltpu.async_copy(
            word_hbm.at[idx_v.at[c + 1]], fbufs[(c + 1) & 1], gsems[(c + 1) & 1])
      if c >= 2:
        sh[c - 2].wait()
      pack_chunk(fbufs[b], pbufs[b])
      sh[c] = pltpu.async_copy(
          pbufs[b], out_hbm.at[pl.ds(base + c * CHUNK, CHUNK)], osems[b])
    sh[NCHUNK - 2].wait()
    sh[NCHUNK - 1].wait()

  return sc_gather


SEQ_BLK = 2


def _tc_ln_kernel(g_ref, pos_ref, tt_ref, type_ref, scale_ref, bias_ref,
                  out_ref, *rest):
  w = g_ref[...].reshape(SEQ_BLK, S, HW)
  xa = lax.bitcast_convert_type(w << 16, jnp.float32)
  xb = lax.bitcast_convert_type(w & jnp.int32(-65536), jnp.float32)
  pos = pos_ref[...]
  tt = tt_ref[...]
  t0 = type_ref[0, :]
  t1 = type_ref[1, :]
  typa = jnp.where(tt == 1.0, t1[None, None, :HW], t0[None, None, :HW])
  typb = jnp.where(tt == 1.0, t1[None, None, HW:], t0[None, None, HW:])
  ha = xa + pos[None, :, :HW] + typa
  hb = xb + pos[None, :, HW:] + typb
  ssum = (jnp.sum(ha, axis=-1, keepdims=True)
          + jnp.sum(hb, axis=-1, keepdims=True))
  ssq = (jnp.sum(ha * ha, axis=-1, keepdims=True)
         + jnp.sum(hb * hb, axis=-1, keepdims=True))
  mean = ssum * (1.0 / H)
  var = ssq * (1.0 / H) - mean * mean
  r = lax.rsqrt(var + EPS)
  out_ref[:, :, :HW] = (ha - mean) * r * scale_ref[:, :HW] + bias_ref[:, :HW]
  out_ref[:, :, HW:] = (hb - mean) * r * scale_ref[:, HW:] + bias_ref[:, HW:]


def _tc_ln_first_kernel(g_ref, pos_ref, tt_ref, type_ref, scale_ref,
                        bias_ref, out_ref):
  _tc_ln_kernel(g_ref, pos_ref, tt_ref, type_ref, scale_ref, bias_ref,
                out_ref)


def _tc_ln_acc_kernel(g_ref, pos_ref, tt_ref, type_ref, scale_ref,
                      bias_ref, o_prev_ref, out_ref):
  _tc_ln_kernel(g_ref, pos_ref, tt_ref, type_ref, scale_ref, bias_ref,
                out_ref)


@functools.lru_cache(maxsize=None)
def _tc_ln_call(seq_off, first):
  blk_off = seq_off // SEQ_BLK
  in_specs = [
      pl.BlockSpec((SEQ_BLK * S, HW), lambda b: (b, 0)),
      pl.BlockSpec((S, H), lambda b: (0, 0)),
      pl.BlockSpec((SEQ_BLK, S, 1), lambda b: (b, 0, 0)),
      pl.BlockSpec((2, H), lambda b: (0, 0)),
      pl.BlockSpec((1, H), lambda b: (0, 0)),
      pl.BlockSpec((1, H), lambda b: (0, 0)),
  ]
  kwargs = {}
  if first:
    body = _tc_ln_first_kernel
  else:
    body = _tc_ln_acc_kernel
    in_specs = in_specs + [pl.BlockSpec(memory_space=pltpu.MemorySpace.HBM)]
    kwargs["input_output_aliases"] = {6: 0}
  return pl.pallas_call(
      body,
      grid=(SEQ_PER_PIPE // SEQ_BLK,),
      in_specs=in_specs,
      out_specs=pl.BlockSpec((SEQ_BLK, S, H), lambda b: (blk_off + b, 0, 0)),
      out_shape=jax.ShapeDtypeStruct((B, S, H), jnp.float32),
      **kwargs,
  )


def kernel(input_ids, token_type_ids, position_ids, attention_mask,
           word_emb, pos_emb, type_emb, ln_scale, ln_bias):
  del position_ids, attention_mask
  ids = input_ids.astype(jnp.int32).reshape(NPIPE, NW, NCHUNK, CHUNK)
  tt = token_type_ids.astype(jnp.float32).reshape(NPIPE, SEQ_PER_PIPE, S, 1)
  scale2 = ln_scale.reshape(1, H)
  bias2 = ln_bias.reshape(1, H)
  word_i32 = lax.bitcast_convert_type(word_emb, jnp.int32)
  sc = _sc_gather_fn()

  gathered = [sc(word_i32, ids[k]) for k in range(NPIPE)]
  out = None
  for k in range(NPIPE):
    args = (gathered[k], pos_emb, tt[k], type_emb, scale2, bias2)
    if out is None:
      out = _tc_ln_call(0, True)(*args)
    else:
      out = _tc_ln_call(k * SEQ_PER_PIPE, False)(*args, out)
  return out


# speedup vs baseline: 1.5289x; 1.5289x over previous
import functools

import jax
import jax.numpy as jnp
from jax import lax
from jax.experimental import pallas as pl
from jax.experimental.pallas import tpu as pltpu
from jax.experimental.pallas import tpu_sc as plsc

B, S, H = 64, 512, 768
V = 30522
EPS = 1e-12

NC, NS = 2, 16
NW = NC * NS
TOK = B * S
CHUNK = 32
NPIPE = 8
SEQ_PER_PIPE = B // NPIPE
TOK_PER_PIPE = TOK // NPIPE
TPW = TOK_PER_PIPE // NW
NCHUNK = TPW // CHUNK


@functools.lru_cache(maxsize=1)
def _sc_gather_fn():
  mesh = plsc.VectorSubcoreMesh(core_axis_name="c", subcore_axis_name="s",
                                num_cores=NC, num_subcores=NS)

  @functools.partial(
      pl.kernel,
      mesh=mesh,
      out_type=jax.ShapeDtypeStruct((TOK_PER_PIPE, H), jnp.float32),
      scratch_types=[
          pltpu.VMEM((NCHUNK, CHUNK), jnp.int32),
          pltpu.VMEM((CHUNK, H), jnp.float32),
          pltpu.VMEM((CHUNK, H), jnp.float32),
          pltpu.SemaphoreType.DMA,
          pltpu.SemaphoreType.DMA,
          pltpu.SemaphoreType.DMA,
          pltpu.SemaphoreType.DMA,
      ],
  )
  def sc_gather(word_hbm, ids_hbm, out_hbm, idx_v, r0, r1, g0, g1, o0, o1):
    wid = lax.axis_index("s") * NC + lax.axis_index("c")
    base = wid * TPW
    bufs = (r0, r1)
    gsems = (g0, g1)
    osems = (o0, o1)
    pltpu.sync_copy(ids_hbm.at[wid], idx_v)

    gh = [None] * NCHUNK
    sh = [None] * NCHUNK
    gh[0] = pltpu.async_copy(word_hbm.at[idx_v.at[0]], bufs[0], gsems[0])
    for c in range(NCHUNK):
      b = c & 1
      gh[c].wait()
      if c + 1 < NCHUNK:
        nb = (c + 1) & 1
        if c >= 1:
          sh[c - 1].wait()
        gh[c + 1] = pltpu.async_copy(
            word_hbm.at[idx_v.at[c + 1]], bufs[nb], gsems[nb])
      sh[c] = pltpu.async_copy(
          bufs[b], out_hbm.at[pl.ds(base + c * CHUNK, CHUNK)], osems[b])
    if NCHUNK >= 2:
      sh[NCHUNK - 2].wait()
    sh[NCHUNK - 1].wait()

  return sc_gather


SEQ_BLK = 2


def _tc_ln_kernel(g_ref, pos_ref, tt_ref, type_ref, scale_ref, bias_ref,
                  out_ref, *rest):
  x = g_ref[...].reshape(SEQ_BLK, S, H)
  pos = pos_ref[...]
  tt = tt_ref[...]
  t0 = type_ref[0, :]
  t1 = type_ref[1, :]
  typ = jnp.where(tt == 1.0, t1[None, None, :], t0[None, None, :])
  h = x + pos[None] + typ
  mean = jnp.mean(h, axis=-1, keepdims=True)
  var = jnp.mean(h * h, axis=-1, keepdims=True) - mean * mean
  normed = (h - mean) * lax.rsqrt(var + EPS)
  out_ref[...] = normed * scale_ref[...] + bias_ref[...]


def _tc_ln_first_kernel(g_ref, pos_ref, tt_ref, type_ref, scale_ref,
                        bias_ref, out_ref):
  _tc_ln_kernel(g_ref, pos_ref, tt_ref, type_ref, scale_ref, bias_ref,
                out_ref)


def _tc_ln_acc_kernel(g_ref, pos_ref, tt_ref, type_ref, scale_ref,
                      bias_ref, o_prev_ref, out_ref):
  _tc_ln_kernel(g_ref, pos_ref, tt_ref, type_ref, scale_ref, bias_ref,
                out_ref)


@functools.lru_cache(maxsize=None)
def _tc_ln_call(seq_off, first):
  blk_off = seq_off // SEQ_BLK
  in_specs = [
      pl.BlockSpec((SEQ_BLK * S, H), lambda b: (b, 0)),
      pl.BlockSpec((S, H), lambda b: (0, 0)),
      pl.BlockSpec((SEQ_BLK, S, 1), lambda b: (b, 0, 0)),
      pl.BlockSpec((2, H), lambda b: (0, 0)),
      pl.BlockSpec((1, H), lambda b: (0, 0)),
      pl.BlockSpec((1, H), lambda b: (0, 0)),
  ]
  kwargs = {}
  if first:
    body = _tc_ln_first_kernel
  else:
    body = _tc_ln_acc_kernel
    in_specs = in_specs + [pl.BlockSpec(memory_space=pltpu.MemorySpace.HBM)]
    kwargs["input_output_aliases"] = {6: 0}
  return pl.pallas_call(
      body,
      grid=(SEQ_PER_PIPE // SEQ_BLK,),
      in_specs=in_specs,
      out_specs=pl.BlockSpec((SEQ_BLK, S, H), lambda b: (blk_off + b, 0, 0)),
      out_shape=jax.ShapeDtypeStruct((B, S, H), jnp.float32),
      **kwargs,
  )


def kernel(input_ids, token_type_ids, position_ids, attention_mask,
           word_emb, pos_emb, type_emb, ln_scale, ln_bias):
  del position_ids, attention_mask
  ids = input_ids.astype(jnp.int32).reshape(NPIPE, NW, NCHUNK, CHUNK)
  tt = token_type_ids.astype(jnp.float32).reshape(NPIPE, SEQ_PER_PIPE, S, 1)
  scale2 = ln_scale.reshape(1, H)
  bias2 = ln_bias.reshape(1, H)
  sc = _sc_gather_fn()

  gathered = [sc(word_emb, ids[k]) for k in range(NPIPE)]
  out = None
  for k in range(NPIPE):
    args = (gathered[k], pos_emb, tt[k], type_emb, scale2, bias2)
    if out is None:
      out = _tc_ln_call(0, True)(*args)
    else:
      out = _tc_ln_call(k * SEQ_PER_PIPE, False)(*args, out)
  return out


# speedup vs baseline: 1.7211x; 1.1257x over previous
import functools

import jax
import jax.numpy as jnp
from jax import lax
from jax.experimental import pallas as pl
from jax.experimental.pallas import tpu as pltpu
from jax.experimental.pallas import tpu_sc as plsc

B, S, H = 64, 512, 768
V = 30522
EPS = 1e-12

NC, NS = 2, 16
NW = NC * NS
TOK = B * S
CHUNK = 64
NPIPE = 4
SEQ_PER_PIPE = B // NPIPE
TOK_PER_PIPE = TOK // NPIPE
TPW = TOK_PER_PIPE // NW
NCHUNK = TPW // CHUNK


@functools.lru_cache(maxsize=1)
def _sc_gather_fn():
  mesh = plsc.VectorSubcoreMesh(core_axis_name="c", subcore_axis_name="s",
                                num_cores=NC, num_subcores=NS)

  @functools.partial(
      pl.kernel,
      mesh=mesh,
      out_type=jax.ShapeDtypeStruct((TOK_PER_PIPE, H), jnp.float32),
      scratch_types=[
          pltpu.VMEM((NCHUNK, CHUNK), jnp.int32),
          pltpu.VMEM((CHUNK, H), jnp.float32),
          pltpu.VMEM((CHUNK, H), jnp.float32),
          pltpu.SemaphoreType.DMA,
          pltpu.SemaphoreType.DMA,
          pltpu.SemaphoreType.DMA,
          pltpu.SemaphoreType.DMA,
      ],
  )
  def sc_gather(word_hbm, ids_hbm, out_hbm, idx_v, r0, r1, g0, g1, o0, o1):
    wid = lax.axis_index("s") * NC + lax.axis_index("c")
    base = wid * TPW
    bufs = (r0, r1)
    gsems = (g0, g1)
    osems = (o0, o1)
    pltpu.sync_copy(ids_hbm.at[wid], idx_v)

    gh = [None] * NCHUNK
    sh = [None] * NCHUNK
    gh[0] = pltpu.async_copy(word_hbm.at[idx_v.at[0]], bufs[0], gsems[0])
    for c in range(NCHUNK):
      b = c & 1
      gh[c].wait()
      if c + 1 < NCHUNK:
        nb = (c + 1) & 1
        if c >= 1:
          sh[c - 1].wait()
        gh[c + 1] = pltpu.async_copy(
            word_hbm.at[idx_v.at[c + 1]], bufs[nb], gsems[nb])
      sh[c] = pltpu.async_copy(
          bufs[b], out_hbm.at[pl.ds(base + c * CHUNK, CHUNK)], osems[b])
    if NCHUNK >= 2:
      sh[NCHUNK - 2].wait()
    sh[NCHUNK - 1].wait()

  return sc_gather


SEQ_BLK = 4


def _tc_ln_kernel(g_ref, pos_ref, tt_ref, type_ref, scale_ref, bias_ref,
                  out_ref, *rest):
  x = g_ref[...].reshape(SEQ_BLK, S, H)
  pos = pos_ref[...]
  tt = tt_ref[...]
  t0 = type_ref[0, :]
  t1 = type_ref[1, :]
  typ = jnp.where(tt == 1.0, t1[None, None, :], t0[None, None, :])
  h = x + pos[None] + typ
  mean = jnp.mean(h, axis=-1, keepdims=True)
  var = jnp.mean(h * h, axis=-1, keepdims=True) - mean * mean
  normed = (h - mean) * lax.rsqrt(var + EPS)
  out_ref[...] = normed * scale_ref[...] + bias_ref[...]


def _tc_ln_first_kernel(g_ref, pos_ref, tt_ref, type_ref, scale_ref,
                        bias_ref, out_ref):
  _tc_ln_kernel(g_ref, pos_ref, tt_ref, type_ref, scale_ref, bias_ref,
                out_ref)


def _tc_ln_acc_kernel(g_ref, pos_ref, tt_ref, type_ref, scale_ref,
                      bias_ref, o_prev_ref, out_ref):
  _tc_ln_kernel(g_ref, pos_ref, tt_ref, type_ref, scale_ref, bias_ref,
                out_ref)


@functools.lru_cache(maxsize=None)
def _tc_ln_call(seq_off, first):
  blk_off = seq_off // SEQ_BLK
  in_specs = [
      pl.BlockSpec((SEQ_BLK * S, H), lambda b: (b, 0)),
      pl.BlockSpec((S, H), lambda b: (0, 0)),
      pl.BlockSpec((SEQ_BLK, S, 1), lambda b: (b, 0, 0)),
      pl.BlockSpec((2, H), lambda b: (0, 0)),
      pl.BlockSpec((1, H), lambda b: (0, 0)),
      pl.BlockSpec((1, H), lambda b: (0, 0)),
  ]
  kwargs = {}
  if first:
    body = _tc_ln_first_kernel
  else:
    body = _tc_ln_acc_kernel
    in_specs = in_specs + [pl.BlockSpec(memory_space=pltpu.MemorySpace.HBM)]
    kwargs["input_output_aliases"] = {6: 0}
  return pl.pallas_call(
      body,
      grid=(SEQ_PER_PIPE // SEQ_BLK,),
      in_specs=in_specs,
      out_specs=pl.BlockSpec((SEQ_BLK, S, H), lambda b: (blk_off + b, 0, 0)),
      out_shape=jax.ShapeDtypeStruct((B, S, H), jnp.float32),
      **kwargs,
  )


def kernel(input_ids, token_type_ids, position_ids, attention_mask,
           word_emb, pos_emb, type_emb, ln_scale, ln_bias):
  del position_ids, attention_mask
  ids = input_ids.astype(jnp.int32).reshape(NPIPE, NW, NCHUNK, CHUNK)
  tt = token_type_ids.astype(jnp.float32).reshape(NPIPE, SEQ_PER_PIPE, S, 1)
  scale2 = ln_scale.reshape(1, H)
  bias2 = ln_bias.reshape(1, H)
  sc = _sc_gather_fn()

  gathered = [sc(word_emb, ids[k]) for k in range(NPIPE)]
  out = None
  for k in range(NPIPE):
    args = (gathered[k], pos_emb, tt[k], type_emb, scale2, bias2)
    if out is None:
      out = _tc_ln_call(0, True)(*args)
    else:
      out = _tc_ln_call(k * SEQ_PER_PIPE, False)(*args, out)
  return out
